# baseline (speedup 1.0000x reference)
import jax
import jax.numpy as jnp
from jax import lax
from jax.experimental import pallas as pl
from jax.experimental.pallas import tpu as pltpu
from jax.experimental.pallas import tpu_sc as plsc

B = 1024
D = 512
NUM_CLASS = 100000

NW = 32
BPW = B // NW
CH = 8
NCH = BPW // CH
WV = CH // 2


def _sc_body(label_ref, cls_ref, cloth_ref, tmpl_ref, tail_ref, out_ref,
             idx_v, g_v, hd_v, tail_v, sem_g, sem_h, sem_s, sem_l):
    wid = lax.axis_index("s") * 2 + lax.axis_index("c")
    base = wid * BPW

    pltpu.sync_copy(label_ref.at[pl.ds(base, BPW)], idx_v)
    pltpu.async_copy(tail_ref, tail_v, sem_s)
    for r in range(CH):
        pltpu.async_copy(tmpl_ref, hd_v.at[r], sem_l)
    pltpu.make_async_copy(tail_ref, tail_v, sem_s).wait()
    for r in range(CH):
        pltpu.make_async_copy(tmpl_ref, hd_v.at[r], sem_l).wait()

    @pl.loop(0, NCH)
    def _(c):
        crow = base + c * CH

        @pl.when(c >= 1)
        def _():
            for w in range(2):
                pltpu.make_async_copy(
                    hd_v.at[pl.ds(w * WV, WV)],
                    out_ref.at[pl.ds(crow - CH + w * WV, WV), pl.ds(0, 16)],
                    sem_h).wait()

        pltpu.async_copy(cls_ref.at[idx_v.at[pl.ds(c * CH, CH)]], g_v,
                         sem_g).wait()

        @pl.loop(0, CH)
        def _(r):
            for t in range(4):
                for v in range(D // 16):
                    sl = pl.ds(v * 16, 16)
                    hd_v[r, 5 + t, sl] = g_v[r, t, sl]

        pltpu.async_copy(cloth_ref.at[idx_v.at[pl.ds(c * CH, CH)]], g_v,
                         sem_g).wait()

        @pl.loop(0, CH)
        def _(r):
            for t in range(4):
                for v in range(D // 16):
                    sl = pl.ds(v * 16, 16)
                    hd_v[r, 11 + t, sl] = g_v[r, t, sl]

        for w in range(2):
            pltpu.async_copy(
                hd_v.at[pl.ds(w * WV, WV)],
                out_ref.at[pl.ds(crow + w * WV, WV), pl.ds(0, 16)], sem_h)

        @pl.loop(0, CH)
        def _(r):
            pltpu.async_copy(tail_v, out_ref.at[crow + r, pl.ds(16, 56)],
                             sem_s)

    for w in range(2):
        pltpu.make_async_copy(
            hd_v.at[pl.ds(w * WV, WV)],
            out_ref.at[pl.ds(base + (NCH - 1) * CH + w * WV, WV),
                       pl.ds(0, 16)],
            sem_h).wait()

    @pl.loop(0, BPW)
    def _(i):
        pltpu.make_async_copy(tail_v, out_ref.at[base + i, pl.ds(16, 56)],
                              sem_s).wait()


def kernel(label, cls_ctx, cloth_cls_ctx, token_prefix, token_mid, token_suffix):
    lbl = label.astype(jnp.int32)
    pre = token_prefix.reshape(5, D)
    mid = token_mid.reshape(2, D)
    suf = token_suffix.reshape(62, D)
    tmpl = jnp.concatenate(
        [pre, jnp.zeros((4, D), jnp.float32), mid,
         jnp.zeros((4, D), jnp.float32), suf[:1]], axis=0)
    tail = suf[1:57]

    out_ref = jax.new_ref(lax.empty((B, 77, D), jnp.float32))
    mesh = plsc.VectorSubcoreMesh(core_axis_name="c", subcore_axis_name="s")
    pl.kernel(
        _sc_body,
        out_type=(),
        mesh=mesh,
        scratch_types=[
            pltpu.VMEM((BPW,), jnp.int32),
            pltpu.VMEM((CH, 4, D), jnp.float32),
            pltpu.VMEM((CH, 16, D), jnp.float32),
            pltpu.VMEM((56, D), jnp.float32),
            pltpu.SemaphoreType.DMA,
            pltpu.SemaphoreType.DMA,
            pltpu.SemaphoreType.DMA,
            pltpu.SemaphoreType.DMA,
        ],
    )(lbl, cls_ctx, cloth_cls_ctx, tmpl, tail, out_ref)
    out = out_ref[...]
    tail5 = jnp.broadcast_to(suf[57:62], (B, 5, D))
    return lax.dynamic_update_slice(out, tail5, (0, 72, 0))


# speedup vs baseline: 1.0538x; 1.0538x over previous
import jax
import jax.numpy as jnp
from jax import lax
from jax.experimental import pallas as pl
from jax.experimental.pallas import tpu as pltpu
from jax.experimental.pallas import tpu_sc as plsc

B = 1024
D = 512
NUM_CLASS = 100000

NW = 32
BPW = B // NW
CH = 8
NCH = BPW // CH
WV = CH // 2


def _sc_body(label_ref, cls_ref, cloth_ref, tmpl_ref, tail_ref, out_ref,
             idx_v, g_v, hd_v, tail_v, sem_g, sem_h, sem_s, sem_l):
    wid = lax.axis_index("s") * 2 + lax.axis_index("c")
    base = wid * BPW

    pltpu.sync_copy(label_ref.at[pl.ds(base, BPW)], idx_v)
    pltpu.async_copy(tail_ref, tail_v, sem_s)
    for r in range(CH):
        pltpu.async_copy(tmpl_ref, hd_v.at[r], sem_l)
    pltpu.make_async_copy(tail_ref, tail_v, sem_s).wait()
    for r in range(CH):
        pltpu.make_async_copy(tmpl_ref, hd_v.at[r], sem_l).wait()

    @pl.loop(0, NCH)
    def _(c):
        crow = base + c * CH

        @pl.when(c >= 1)
        def _():
            for w in range(2):
                pltpu.make_async_copy(
                    hd_v.at[pl.ds(w * WV, WV)],
                    out_ref.at[pl.ds(crow - CH + w * WV, WV), pl.ds(0, 16)],
                    sem_h).wait()

        pltpu.async_copy(cls_ref.at[idx_v.at[pl.ds(c * CH, CH)]], g_v,
                         sem_g).wait()

        @pl.loop(0, CH)
        def _(r):
            for t in range(4):
                for v in range(D // 16):
                    sl = pl.ds(v * 16, 16)
                    hd_v[r, 5 + t, sl] = g_v[r, t, sl]

        pltpu.async_copy(cloth_ref.at[idx_v.at[pl.ds(c * CH, CH)]], g_v,
                         sem_g).wait()

        @pl.loop(0, CH)
        def _(r):
            for t in range(4):
                for v in range(D // 16):
                    sl = pl.ds(v * 16, 16)
                    hd_v[r, 11 + t, sl] = g_v[r, t, sl]

        for w in range(2):
            pltpu.async_copy(
                hd_v.at[pl.ds(w * WV, WV)],
                out_ref.at[pl.ds(crow + w * WV, WV), pl.ds(0, 16)], sem_h)

        @pl.loop(0, CH)
        def _(r):
            pltpu.async_copy(tail_v, out_ref.at[crow + r, pl.ds(16, 56)],
                             sem_s)

    for w in range(2):
        pltpu.make_async_copy(
            hd_v.at[pl.ds(w * WV, WV)],
            out_ref.at[pl.ds(base + (NCH - 1) * CH + w * WV, WV),
                       pl.ds(0, 16)],
            sem_h).wait()

    @pl.loop(0, BPW)
    def _(i):
        pltpu.make_async_copy(tail_v, out_ref.at[base + i, pl.ds(16, 56)],
                              sem_s).wait()


def kernel(label, cls_ctx, cloth_cls_ctx, token_prefix, token_mid, token_suffix):
    lbl = label.astype(jnp.int32)
    pre = token_prefix.reshape(5, D)
    mid = token_mid.reshape(2, D)
    suf = token_suffix.reshape(62, D)
    tmpl = jnp.concatenate(
        [pre, jnp.zeros((4, D), jnp.float32), mid,
         jnp.zeros((4, D), jnp.float32), suf[:1]], axis=0)
    tail = suf[1:57]

    out_ref = jax.new_ref(lax.empty((B, 77, D), jnp.float32))
    out_ref[:, 72:77, :] = jnp.broadcast_to(suf[57:62], (B, 5, D))
    mesh = plsc.VectorSubcoreMesh(core_axis_name="c", subcore_axis_name="s")
    pl.kernel(
        _sc_body,
        out_type=(),
        mesh=mesh,
        scratch_types=[
            pltpu.VMEM((BPW,), jnp.int32),
            pltpu.VMEM((CH, 4, D), jnp.float32),
            pltpu.VMEM((CH, 16, D), jnp.float32),
            pltpu.VMEM((56, D), jnp.float32),
            pltpu.SemaphoreType.DMA,
            pltpu.SemaphoreType.DMA,
            pltpu.SemaphoreType.DMA,
            pltpu.SemaphoreType.DMA,
        ],
    )(lbl, cls_ctx, cloth_cls_ctx, tmpl, tail, out_ref)
    return out_ref[...]


# speedup vs baseline: 1.1996x; 1.1384x over previous
import functools

import jax
import jax.numpy as jnp
from jax import lax
from jax.experimental import pallas as pl
from jax.experimental.pallas import tpu as pltpu
from jax.experimental.pallas import tpu_sc as plsc

B = 1024
D = 512
NUM_CLASS = 100000

NW = 32
BPW = B // NW
BR = 64
GRID = B // BR


def _sc_gather(label_ref, cls_ref, cloth_ref, gc_ref, gl_ref,
               idx_v, ga_v, sem_g, sem_w):
    wid = lax.axis_index("s") * 2 + lax.axis_index("c")
    base = wid * BPW

    pltpu.sync_copy(label_ref.at[pl.ds(base, BPW)], idx_v)

    pltpu.async_copy(cls_ref.at[idx_v], ga_v, sem_g).wait()
    pltpu.async_copy(ga_v, gc_ref.at[pl.ds(base, BPW)], sem_w)
    pltpu.make_async_copy(ga_v, gc_ref.at[pl.ds(base, BPW)], sem_w).wait()

    pltpu.async_copy(cloth_ref.at[idx_v], ga_v, sem_g).wait()
    pltpu.async_copy(ga_v, gl_ref.at[pl.ds(base, BPW)], sem_w)
    pltpu.make_async_copy(ga_v, gl_ref.at[pl.ds(base, BPW)], sem_w).wait()


def _tc_assemble(gc_ref, gl_ref, pre_ref, mid_ref, suf_ref, o_ref):
    for t in range(5):
        o_ref[:, t, :] = jnp.broadcast_to(pre_ref[t], (BR, D))
    for t in range(4):
        o_ref[:, 5 + t, :] = gc_ref[:, t, :]
    for t in range(2):
        o_ref[:, 9 + t, :] = jnp.broadcast_to(mid_ref[t], (BR, D))
    for t in range(4):
        o_ref[:, 11 + t, :] = gl_ref[:, t, :]
    for t in range(62):
        o_ref[:, 15 + t, :] = jnp.broadcast_to(suf_ref[t], (BR, D))


def kernel(label, cls_ctx, cloth_cls_ctx, token_prefix, token_mid, token_suffix):
    lbl = label.astype(jnp.int32)

    mesh = plsc.VectorSubcoreMesh(core_axis_name="c", subcore_axis_name="s")
    gc, gl = pl.kernel(
        _sc_gather,
        out_type=(jax.ShapeDtypeStruct((B, 4, D), jnp.float32),
                  jax.ShapeDtypeStruct((B, 4, D), jnp.float32)),
        mesh=mesh,
        scratch_types=[
            pltpu.VMEM((BPW,), jnp.int32),
            pltpu.VMEM((BPW, 4, D), jnp.float32),
            pltpu.SemaphoreType.DMA,
            pltpu.SemaphoreType.DMA,
        ],
    )(lbl, cls_ctx, cloth_cls_ctx)

    out = pl.pallas_call(
        _tc_assemble,
        out_shape=jax.ShapeDtypeStruct((B, 77, D), jnp.float32),
        grid=(GRID,),
        in_specs=[
            pl.BlockSpec((BR, 4, D), lambda i: (i, 0, 0)),
            pl.BlockSpec((BR, 4, D), lambda i: (i, 0, 0)),
            pl.BlockSpec((5, D), lambda i: (0, 0)),
            pl.BlockSpec((2, D), lambda i: (0, 0)),
            pl.BlockSpec((62, D), lambda i: (0, 0)),
        ],
        out_specs=pl.BlockSpec((BR, 77, D), lambda i: (i, 0, 0)),
        compiler_params=pltpu.CompilerParams(
            dimension_semantics=("arbitrary",)),
    )(gc, gl, token_prefix.reshape(5, D), token_mid.reshape(2, D),
      token_suffix.reshape(62, D))
    return out
